# baseline (speedup 1.0000x reference)
import functools
import math

import jax
import jax.numpy as jnp
import numpy as np
from jax.experimental import pallas as pl
from jax.experimental.pallas import tpu as pltpu

_N_SPIXELS = 256
_N_ITERS = 5
_NEG = -1e16
_L2E = 1.4426950408889634


def _cells_layout(h, w, n_spixels):
    nw = int(math.sqrt(n_spixels * w / h) + 0.5)
    nh = int(math.sqrt(n_spixels * h / w) + 0.5)
    cw = int(math.ceil(w / nw))
    ch = int(math.ceil(h / nh))
    return nh, nw, ch, cw


def _consts(h, w, nh, nw, ch, cw, b, c):
    j_of_l = np.minimum(np.arange(w) // cw, nw - 1)
    E = np.zeros((3, w, nw), np.float32)
    cbias = np.zeros((3, 1, w), np.float32)
    for t, dx in enumerate((-1, 0, 1)):
        jj = j_of_l + dx
        ok = (jj >= 0) & (jj < nw)
        jc = np.clip(jj, 0, nw - 1)
        E[t, np.arange(w), jc] = 1.0
        cbias[t, 0] = np.where(ok, 0.0, _NEG).astype(np.float32)
    G = np.ascontiguousarray(np.transpose(E, (0, 2, 1)))
    R = np.zeros((b, b * c), np.float32)
    for bi in range(b):
        R[bi, bi * c:(bi + 1) * c] = 1.0
    R2L = np.zeros((b * c, b), np.float32)
    for g in range(b * c):
        R2L[g, g // c] = _L2E
    Gn8 = np.zeros((b * c * (ch // 3), b * c), np.float32)
    for g in range(b * c * (ch // 3)):
        Gn8[g, g // (ch // 3)] = 1.0
    Gd8 = np.zeros((b * (ch // 3), b), np.float32)
    for g in range(b * (ch // 3)):
        Gd8[g, g // (ch // 3)] = 1.0
    return tuple(jnp.asarray(a) for a in (E, G, cbias, R, R2L, Gn8, Gd8))


def _scat(e_k, col):
    return jax.lax.dot_general(e_k, col, (((0,), (1,)), ((), ())),
                               preferred_element_type=jnp.float32)


def _mm(a, bmat):
    return jax.lax.dot_general(a, bmat, (((1,), (0,)), ((), ())),
                               preferred_element_type=jnp.float32)


def _collapse3(arr, g, ch, w):
    return arr.reshape(g, 3, ch // 3, w).sum(axis=1).reshape(g * (ch // 3), w)


def _ssn_body(x_ref, e_ref, g_ref, cbias_ref, rrep_ref, r2l_ref, gn8_ref,
              gd8_ref, q_ref, spfp_ref, spf_s, num_s, den_s,
              *, nh, nw, ch, b, c, n_iters):
    i = pl.program_id(0)
    r = pl.program_id(1)
    w = x_ref.shape[-1]
    bc = b * c
    X = x_ref[...]
    X20 = X.reshape(bc, ch, w)

    @pl.when(jnp.logical_and(i == 0, r == 0))
    def _zero():
        num_s[...] = jnp.zeros_like(num_s)
        den_s[...] = jnp.zeros_like(den_s)

    @pl.when(i == 0)
    def _init():
        col8 = _collapse3(X20, bc, ch, w)
        cn = _mm(_scat(e_ref[1], col8), gn8_ref[...])
        num_s[pl.ds(r * nw, nw), :] += cn
        cnt = jnp.full((b, w), float(ch), jnp.float32)
        den_s[pl.ds(r * nw, nw), :] += _scat(e_ref[1], cnt)

    @pl.when(i > 0)
    def _iterate():
        xsqL = _L2E * jnp.sum(X * X, axis=1)
        nd = []
        for dy in (-1, 0, 1):
            rn = r + dy
            rbias = jnp.where(jnp.logical_and(rn >= 0, rn < nh),
                              jnp.float32(0.0), jnp.float32(_NEG))
            rp = jnp.clip(rn, 0, nh - 1)
            S = spf_s[pl.ds(rp * nw, nw), :]
            Saug = jnp.concatenate(
                [(2.0 * _L2E) * S, _mm(S * S, r2l_ref[...])], axis=1)
            for t_dx in range(3):
                MapA = jax.lax.dot_general(
                    Saug, g_ref[t_dx], (((0,), (0,)), ((), ())),
                    preferred_element_type=jnp.float32)
                M2 = MapA[:bc].reshape(b, c, 1, w)
                ssqL = MapA[bc:bc + b]
                baserow = (cbias_ref[t_dx] + rbias) - ssqL
                cr = jnp.sum(X * M2, axis=1)
                nd.append((baserow[:, None, :] - xsqL) + cr)
        m = nd[0]
        for t in range(1, 9):
            m = jnp.maximum(m, nd[t])
        ex = [jnp.exp2(nd[t] - m) for t in range(9)]
        s = ex[0]
        for t in range(1, 9):
            s = s + ex[t]
        rs = 1.0 / s
        Xn = X * rs[:, None]

        @pl.when(i == n_iters)
        def _emit_q():
            q_ref[...] = jnp.stack([ex[t] * rs for t in range(9)], axis=1)

        for t_dy, dy in enumerate((-1, 0, 1)):
            rp = jnp.clip(r + dy, 0, nh - 1)
            an = jnp.zeros((nw, bc * (ch // 3)), jnp.float32)
            ad = jnp.zeros((nw, b * (ch // 3)), jnp.float32)
            for t_dx in range(3):
                k = t_dy * 3 + t_dx
                P8 = _collapse3(ex[k][:, None] * Xn, bc, ch, w)
                q8 = _collapse3(ex[k] * rs, b, ch, w)
                an = an + _scat(e_ref[t_dx], P8)
                ad = ad + _scat(e_ref[t_dx], q8)
            num_s[pl.ds(rp * nw, nw), :] += _mm(an, gn8_ref[...])
            den_s[pl.ds(rp * nw, nw), :] += _mm(ad, gd8_ref[...])

    @pl.when(r == nh - 1)
    def _finalize():
        den = den_s[...]
        den_bc = jax.lax.dot_general(
            den, rrep_ref[...], (((1,), (0,)), ((), ())),
            preferred_element_type=jnp.float32)
        denom = jnp.where(i == 0, jnp.maximum(den_bc, 1.0), den_bc + 1e-16)
        spf = num_s[...] / denom
        spf_s[...] = spf
        num_s[...] = jnp.zeros_like(num_s)
        den_s[...] = jnp.zeros_like(den_s)

        @pl.when(i == n_iters)
        def _emit_spf():
            spfp_ref[...] = spf


@jax.jit
def kernel(x):
    b, c, h, w = x.shape
    nh, nw, ch, cw = _cells_layout(h, w, _N_SPIXELS)
    assert nh * ch == h and nw * cw == w, "kernel assumes even cell tiling"
    assert ch % 3 == 0
    n_sp = nh * nw
    E, G, cbias, R, R2L, Gn8, Gd8 = _consts(h, w, nh, nw, ch, cw, b, c)
    grid = (_N_ITERS + 1, nh)
    body = functools.partial(_ssn_body, nh=nh, nw=nw, ch=ch, b=b, c=c,
                             n_iters=_N_ITERS)
    q, spf_p = pl.pallas_call(
        body,
        grid=grid,
        in_specs=[
            pl.BlockSpec((b, c, ch, w), lambda i, r: (0, 0, r, 0)),
            pl.BlockSpec((3, w, nw), lambda i, r: (0, 0, 0)),
            pl.BlockSpec((3, nw, w), lambda i, r: (0, 0, 0)),
            pl.BlockSpec((3, 1, w), lambda i, r: (0, 0, 0)),
            pl.BlockSpec((b, b * c), lambda i, r: (0, 0)),
            pl.BlockSpec((b * c, b), lambda i, r: (0, 0)),
            pl.BlockSpec((b * c * (ch // 3), b * c), lambda i, r: (0, 0)),
            pl.BlockSpec((b * (ch // 3), b), lambda i, r: (0, 0)),
        ],
        out_specs=[
            pl.BlockSpec((b, 9, ch, w),
                         lambda i, r: (0, 0, jnp.where(i == _N_ITERS, r, 0), 0)),
            pl.BlockSpec((n_sp, b * c), lambda i, r: (0, 0)),
        ],
        out_shape=[
            jax.ShapeDtypeStruct((b, 9, h, w), jnp.float32),
            jax.ShapeDtypeStruct((n_sp, b * c), jnp.float32),
        ],
        scratch_shapes=[
            pltpu.VMEM((n_sp, b * c), jnp.float32),
            pltpu.VMEM((n_sp, b * c), jnp.float32),
            pltpu.VMEM((n_sp, b), jnp.float32),
        ],
        compiler_params=pltpu.CompilerParams(
            dimension_semantics=("arbitrary", "arbitrary")),
    )(x, E, G, cbias, R, R2L, Gn8, Gd8)
    spf_out = spf_p.T.reshape(b, c, n_sp)
    return (q, x, spf_out, x)


# speedup vs baseline: 1.1566x; 1.1566x over previous
import functools
import math

import jax
import jax.numpy as jnp
import numpy as np
from jax.experimental import pallas as pl
from jax.experimental.pallas import tpu as pltpu

_N_SPIXELS = 256
_N_ITERS = 5
_NEG = -1e16
_L2E = 1.4426950408889634


def _cells_layout(h, w, n_spixels):
    nw = int(math.sqrt(n_spixels * w / h) + 0.5)
    nh = int(math.sqrt(n_spixels * h / w) + 0.5)
    cw = int(math.ceil(w / nw))
    ch = int(math.ceil(h / nh))
    return nh, nw, ch, cw


def _consts(h, w, nh, nw, ch, cw, b, c):
    j_of_l = np.minimum(np.arange(w) // cw, nw - 1)
    E = np.zeros((3, w, nw), np.float32)
    cbias = np.zeros((3, 1, w), np.float32)
    for t, dx in enumerate((-1, 0, 1)):
        jj = j_of_l + dx
        ok = (jj >= 0) & (jj < nw)
        jc = np.clip(jj, 0, nw - 1)
        E[t, np.arange(w), jc] = 1.0
        cbias[t, 0] = np.where(ok, 0.0, _NEG).astype(np.float32)
    G = np.ascontiguousarray(np.transpose(E, (0, 2, 1)))
    R = np.zeros((b, b * c), np.float32)
    for bi in range(b):
        R[bi, bi * c:(bi + 1) * c] = 1.0
    R2L = np.zeros((b * c, b), np.float32)
    for g in range(b * c):
        R2L[g, g // c] = _L2E
    Gn8 = np.zeros((b * c * (ch // 3), b * c), np.float32)
    for g in range(b * c * (ch // 3)):
        Gn8[g, g // (ch // 3)] = 1.0
    Gd8 = np.zeros((b * (ch // 3), b), np.float32)
    for g in range(b * (ch // 3)):
        Gd8[g, g // (ch // 3)] = 1.0
    return tuple(jnp.asarray(a) for a in (E, G, cbias, R, R2L, Gn8, Gd8))


def _scat(e_k, col):
    return jax.lax.dot_general(e_k, col, (((0,), (1,)), ((), ())),
                               preferred_element_type=jnp.float32)


def _mm(a, bmat):
    return jax.lax.dot_general(a, bmat, (((1,), (0,)), ((), ())),
                               preferred_element_type=jnp.float32)


def _collapse3(arr, g, ch, w):
    return arr.reshape(g, 3, ch // 3, w).sum(axis=1).reshape(g * (ch // 3), w)


def _ssn_body(x_ref, e_ref, g_ref, cbias_ref, rrep_ref, r2l_ref, gn8_ref,
              gd8_ref, q_ref, spfp_ref, spf_s, num_s, den_s,
              *, nh, nw, ch, b, c, n_iters, rows_per_step):
    i = pl.program_id(0)
    rb = pl.program_id(1)
    w = x_ref.shape[-1]
    bc = b * c
    XB = x_ref[...]

    @pl.when(jnp.logical_and(i == 0, rb == 0))
    def _zero():
        num_s[...] = jnp.zeros_like(num_s)
        den_s[...] = jnp.zeros_like(den_s)

    @pl.when(i == 0)
    def _init():
        for sub in range(rows_per_step):
            r = rb * rows_per_step + sub
            X20 = XB[:, :, sub * ch:(sub + 1) * ch, :].reshape(bc, ch, w)
            col8 = _collapse3(X20, bc, ch, w)
            cn = _mm(_scat(e_ref[1], col8), gn8_ref[...])
            num_s[pl.ds(r * nw, nw), :] += cn
            cnt = jnp.full((b, w), float(ch), jnp.float32)
            den_s[pl.ds(r * nw, nw), :] += _scat(e_ref[1], cnt)

    @pl.when(i > 0)
    def _iterate():
        rps = rows_per_step
        r0 = rb * rps
        maps = {}
        for off in range(rps + 2):
            rn = r0 + off - 1
            rbias = jnp.where(jnp.logical_and(rn >= 0, rn < nh),
                              jnp.float32(0.0), jnp.float32(_NEG))
            rp = jnp.clip(rn, 0, nh - 1)
            S = spf_s[pl.ds(rp * nw, nw), :]
            Saug = jnp.concatenate(
                [(2.0 * _L2E) * S, _mm(S * S, r2l_ref[...])], axis=1)
            for t_dx in range(3):
                MapA = jax.lax.dot_general(
                    Saug, g_ref[t_dx], (((0,), (0,)), ((), ())),
                    preferred_element_type=jnp.float32)
                M2 = MapA[:bc].reshape(b, c, 1, w)
                baserow = (cbias_ref[t_dx] + rbias) - MapA[bc:bc + b]
                maps[(off, t_dx)] = (M2, baserow)

        win_n = {}
        win_d = {}
        for sub in range(rps):
            X = XB[:, :, sub * ch:(sub + 1) * ch, :]
            xsqL = _L2E * jnp.sum(X * X, axis=1)
            nd = []
            for t_dy in range(3):
                for t_dx in range(3):
                    M2, baserow = maps[(sub + t_dy, t_dx)]
                    cr = jnp.sum(X * M2, axis=1)
                    nd.append((baserow[:, None, :] - xsqL) + cr)
            m = nd[0]
            for t in range(1, 9):
                m = jnp.maximum(m, nd[t])
            ex = [jnp.exp2(nd[t] - m) for t in range(9)]
            s = ex[0]
            for t in range(1, 9):
                s = s + ex[t]
            rs = 1.0 / s
            Xn = X * rs[:, None]

            @pl.when(i == n_iters)
            def _emit_q(sub=sub, ex=ex, rs=rs):
                for t in range(9):
                    q_ref[:, t, sub * ch:(sub + 1) * ch, :] = ex[t] * rs

            for t_dy in range(3):
                an = jnp.zeros((nw, bc * (ch // 3)), jnp.float32)
                ad = jnp.zeros((nw, b * (ch // 3)), jnp.float32)
                for t_dx in range(3):
                    k = t_dy * 3 + t_dx
                    P8 = _collapse3(ex[k][:, None] * Xn, bc, ch, w)
                    q8 = _collapse3(ex[k] * rs, b, ch, w)
                    an = an + _scat(e_ref[t_dx], P8)
                    ad = ad + _scat(e_ref[t_dx], q8)
                woff = sub + t_dy
                cn = _mm(an, gn8_ref[...])
                cd = _mm(ad, gd8_ref[...])
                win_n[woff] = cn if woff not in win_n else win_n[woff] + cn
                win_d[woff] = cd if woff not in win_d else win_d[woff] + cd

        interior_n = jnp.concatenate([win_n[o] for o in range(1, rps + 1)],
                                     axis=0)
        interior_d = jnp.concatenate([win_d[o] for o in range(1, rps + 1)],
                                     axis=0)
        num_s[pl.ds(r0 * nw, rps * nw), :] += interior_n
        den_s[pl.ds(r0 * nw, rps * nw), :] += interior_d
        rtop = jnp.clip(r0 - 1, 0, nh - 1)
        num_s[pl.ds(rtop * nw, nw), :] += win_n[0]
        den_s[pl.ds(rtop * nw, nw), :] += win_d[0]
        rbot = jnp.clip(r0 + rps, 0, nh - 1)
        num_s[pl.ds(rbot * nw, nw), :] += win_n[rps + 1]
        den_s[pl.ds(rbot * nw, nw), :] += win_d[rps + 1]

    @pl.when(rb == nh // rows_per_step - 1)
    def _finalize():
        den = den_s[...]
        den_bc = jax.lax.dot_general(
            den, rrep_ref[...], (((1,), (0,)), ((), ())),
            preferred_element_type=jnp.float32)
        denom = jnp.where(i == 0, jnp.maximum(den_bc, 1.0), den_bc + 1e-16)
        spf = num_s[...] / denom
        spf_s[...] = spf
        num_s[...] = jnp.zeros_like(num_s)
        den_s[...] = jnp.zeros_like(den_s)

        @pl.when(i == n_iters)
        def _emit_spf():
            spfp_ref[...] = spf


@jax.jit
def kernel(x):
    b, c, h, w = x.shape
    nh, nw, ch, cw = _cells_layout(h, w, _N_SPIXELS)
    assert nh * ch == h and nw * cw == w, "kernel assumes even cell tiling"
    assert ch % 3 == 0
    n_sp = nh * nw
    E, G, cbias, R, R2L, Gn8, Gd8 = _consts(h, w, nh, nw, ch, cw, b, c)
    rps = 2
    assert nh % rps == 0
    grid = (_N_ITERS + 1, nh // rps)
    body = functools.partial(_ssn_body, nh=nh, nw=nw, ch=ch, b=b, c=c,
                             n_iters=_N_ITERS, rows_per_step=rps)
    q, spf_p = pl.pallas_call(
        body,
        grid=grid,
        in_specs=[
            pl.BlockSpec((b, c, rps * ch, w), lambda i, r: (0, 0, r, 0)),
            pl.BlockSpec((3, w, nw), lambda i, r: (0, 0, 0)),
            pl.BlockSpec((3, nw, w), lambda i, r: (0, 0, 0)),
            pl.BlockSpec((3, 1, w), lambda i, r: (0, 0, 0)),
            pl.BlockSpec((b, b * c), lambda i, r: (0, 0)),
            pl.BlockSpec((b * c, b), lambda i, r: (0, 0)),
            pl.BlockSpec((b * c * (ch // 3), b * c), lambda i, r: (0, 0)),
            pl.BlockSpec((b * (ch // 3), b), lambda i, r: (0, 0)),
        ],
        out_specs=[
            pl.BlockSpec((b, 9, rps * ch, w),
                         lambda i, r: (0, 0, jnp.where(i == _N_ITERS, r, 0), 0)),
            pl.BlockSpec((n_sp, b * c), lambda i, r: (0, 0)),
        ],
        out_shape=[
            jax.ShapeDtypeStruct((b, 9, h, w), jnp.float32),
            jax.ShapeDtypeStruct((n_sp, b * c), jnp.float32),
        ],
        scratch_shapes=[
            pltpu.VMEM((n_sp, b * c), jnp.float32),
            pltpu.VMEM((n_sp, b * c), jnp.float32),
            pltpu.VMEM((n_sp, b), jnp.float32),
        ],
        compiler_params=pltpu.CompilerParams(
            dimension_semantics=("arbitrary", "arbitrary")),
    )(x, E, G, cbias, R, R2L, Gn8, Gd8)
    spf_out = spf_p.T.reshape(b, c, n_sp)
    return (q, x, spf_out, x)


# speedup vs baseline: 1.1839x; 1.0236x over previous
import functools
import math

import jax
import jax.numpy as jnp
import numpy as np
from jax.experimental import pallas as pl
from jax.experimental.pallas import tpu as pltpu

_N_SPIXELS = 256
_N_ITERS = 5
_NEG = -1e16
_L2E = 1.4426950408889634


def _cells_layout(h, w, n_spixels):
    nw = int(math.sqrt(n_spixels * w / h) + 0.5)
    nh = int(math.sqrt(n_spixels * h / w) + 0.5)
    cw = int(math.ceil(w / nw))
    ch = int(math.ceil(h / nh))
    return nh, nw, ch, cw


def _consts(h, w, nh, nw, ch, cw, b, c):
    j_of_l = np.minimum(np.arange(w) // cw, nw - 1)
    E = np.zeros((3, w, nw), np.float32)
    cbias = np.zeros((3, 1, w), np.float32)
    for t, dx in enumerate((-1, 0, 1)):
        jj = j_of_l + dx
        ok = (jj >= 0) & (jj < nw)
        jc = np.clip(jj, 0, nw - 1)
        E[t, np.arange(w), jc] = 1.0
        cbias[t, 0] = np.where(ok, 0.0, _NEG).astype(np.float32)
    G = np.ascontiguousarray(np.transpose(E, (0, 2, 1)))
    R = np.zeros((b, b * c), np.float32)
    for bi in range(b):
        R[bi, bi * c:(bi + 1) * c] = 1.0
    R2L = np.zeros((b * c, b), np.float32)
    for g in range(b * c):
        R2L[g, g // c] = _L2E
    Gn8 = np.zeros((b * c * (ch // 3), b * c), np.float32)
    for g in range(b * c * (ch // 3)):
        Gn8[g, g // (ch // 3)] = 1.0
    Gd8 = np.zeros((b * (ch // 3), b), np.float32)
    for g in range(b * (ch // 3)):
        Gd8[g, g // (ch // 3)] = 1.0
    return tuple(jnp.asarray(a) for a in (E, G, cbias, R, R2L, Gn8, Gd8))


def _scat(e_k, col):
    return jax.lax.dot_general(e_k, col, (((0,), (1,)), ((), ())),
                               preferred_element_type=jnp.float32)


def _mm(a, bmat):
    return jax.lax.dot_general(a, bmat, (((1,), (0,)), ((), ())),
                               preferred_element_type=jnp.float32)


def _collapse3(arr, g, ch, w):
    return arr.reshape(g, 3, ch // 3, w).sum(axis=1).reshape(g * (ch // 3), w)


def _ssn_body(x_ref, e_ref, g_ref, cbias_ref, rrep_ref, r2l_ref, gn8_ref,
              gd8_ref, q_ref, spfp_ref, spf_s, num_s, den_s,
              *, nh, nw, ch, b, c, n_iters, rows_per_step):
    i = pl.program_id(0)
    rb = pl.program_id(1)
    w = x_ref.shape[-1]
    bc = b * c
    XB = x_ref[...]

    @pl.when(jnp.logical_and(i == 0, rb == 0))
    def _zero():
        num_s[...] = jnp.zeros_like(num_s)
        den_s[...] = jnp.zeros_like(den_s)

    @pl.when(i == 0)
    def _init():
        for sub in range(rows_per_step):
            r = rb * rows_per_step + sub
            X20 = XB[:, :, sub * ch:(sub + 1) * ch, :].reshape(bc, ch, w)
            col8 = _collapse3(X20, bc, ch, w)
            cn = _mm(_scat(e_ref[1], col8), gn8_ref[...])
            num_s[pl.ds(r * nw, nw), :] += cn
            cnt = jnp.full((b, w), float(ch), jnp.float32)
            den_s[pl.ds(r * nw, nw), :] += _scat(e_ref[1], cnt)

    @pl.when(i > 0)
    def _iterate():
        rps = rows_per_step
        r0 = rb * rps
        maps = {}
        for off in range(rps + 2):
            rn = r0 + off - 1
            rbias = jnp.where(jnp.logical_and(rn >= 0, rn < nh),
                              jnp.float32(0.0), jnp.float32(_NEG))
            rp = jnp.clip(rn, 0, nh - 1)
            S = spf_s[pl.ds(rp * nw, nw), :]
            Saug = jnp.concatenate(
                [(2.0 * _L2E) * S, _mm(S * S, r2l_ref[...])], axis=1)
            for t_dx in range(3):
                MapA = jax.lax.dot_general(
                    Saug, g_ref[t_dx], (((0,), (0,)), ((), ())),
                    preferred_element_type=jnp.float32)
                M2 = MapA[:bc].reshape(b, c, 1, w)
                baserow = (cbias_ref[t_dx] + rbias) - MapA[bc:bc + b]
                maps[(off, t_dx)] = (M2, baserow)

        win_n = {}
        win_d = {}
        for sub in range(rps):
            X = XB[:, :, sub * ch:(sub + 1) * ch, :]
            xsqL = _L2E * jnp.sum(X * X, axis=1)
            nd = []
            for t_dy in range(3):
                for t_dx in range(3):
                    M2, baserow = maps[(sub + t_dy, t_dx)]
                    cr = jnp.sum(X * M2, axis=1)
                    nd.append((baserow[:, None, :] - xsqL) + cr)
            m = nd[0]
            for t in range(1, 9):
                m = jnp.maximum(m, nd[t])
            ex = [jnp.exp2(nd[t] - m) for t in range(9)]
            s = ex[0]
            for t in range(1, 9):
                s = s + ex[t]
            rs = 1.0 / s
            Xn = X * rs[:, None]

            @pl.when(i == n_iters)
            def _emit_q(sub=sub, ex=ex, rs=rs):
                for t in range(9):
                    q_ref[:, t, sub * ch:(sub + 1) * ch, :] = ex[t] * rs

            for t_dy in range(3):
                an = jnp.zeros((nw, bc * (ch // 3)), jnp.float32)
                ad = jnp.zeros((nw, b * (ch // 3)), jnp.float32)
                for t_dx in range(3):
                    k = t_dy * 3 + t_dx
                    P8 = _collapse3(ex[k][:, None] * Xn, bc, ch, w)
                    q8 = _collapse3(ex[k] * rs, b, ch, w)
                    an = an + _scat(e_ref[t_dx], P8)
                    ad = ad + _scat(e_ref[t_dx], q8)
                woff = sub + t_dy
                cn = _mm(an, gn8_ref[...])
                cd = _mm(ad, gd8_ref[...])
                win_n[woff] = cn if woff not in win_n else win_n[woff] + cn
                win_d[woff] = cd if woff not in win_d else win_d[woff] + cd

        interior_n = jnp.concatenate([win_n[o] for o in range(1, rps + 1)],
                                     axis=0)
        interior_d = jnp.concatenate([win_d[o] for o in range(1, rps + 1)],
                                     axis=0)
        num_s[pl.ds(r0 * nw, rps * nw), :] += interior_n
        den_s[pl.ds(r0 * nw, rps * nw), :] += interior_d
        rtop = jnp.clip(r0 - 1, 0, nh - 1)
        num_s[pl.ds(rtop * nw, nw), :] += win_n[0]
        den_s[pl.ds(rtop * nw, nw), :] += win_d[0]
        rbot = jnp.clip(r0 + rps, 0, nh - 1)
        num_s[pl.ds(rbot * nw, nw), :] += win_n[rps + 1]
        den_s[pl.ds(rbot * nw, nw), :] += win_d[rps + 1]

    @pl.when(rb == nh // rows_per_step - 1)
    def _finalize():
        den = den_s[...]
        den_bc = jax.lax.dot_general(
            den, rrep_ref[...], (((1,), (0,)), ((), ())),
            preferred_element_type=jnp.float32)
        denom = jnp.where(i == 0, jnp.maximum(den_bc, 1.0), den_bc + 1e-16)
        spf = num_s[...] / denom
        spf_s[...] = spf
        num_s[...] = jnp.zeros_like(num_s)
        den_s[...] = jnp.zeros_like(den_s)

        @pl.when(i == n_iters)
        def _emit_spf():
            spfp_ref[...] = spf


@jax.jit
def kernel(x):
    b, c, h, w = x.shape
    nh, nw, ch, cw = _cells_layout(h, w, _N_SPIXELS)
    assert nh * ch == h and nw * cw == w, "kernel assumes even cell tiling"
    assert ch % 3 == 0
    n_sp = nh * nw
    E, G, cbias, R, R2L, Gn8, Gd8 = _consts(h, w, nh, nw, ch, cw, b, c)
    rps = 4
    assert nh % rps == 0
    grid = (_N_ITERS + 1, nh // rps)
    body = functools.partial(_ssn_body, nh=nh, nw=nw, ch=ch, b=b, c=c,
                             n_iters=_N_ITERS, rows_per_step=rps)
    q, spf_p = pl.pallas_call(
        body,
        grid=grid,
        in_specs=[
            pl.BlockSpec((b, c, rps * ch, w), lambda i, r: (0, 0, r, 0)),
            pl.BlockSpec((3, w, nw), lambda i, r: (0, 0, 0)),
            pl.BlockSpec((3, nw, w), lambda i, r: (0, 0, 0)),
            pl.BlockSpec((3, 1, w), lambda i, r: (0, 0, 0)),
            pl.BlockSpec((b, b * c), lambda i, r: (0, 0)),
            pl.BlockSpec((b * c, b), lambda i, r: (0, 0)),
            pl.BlockSpec((b * c * (ch // 3), b * c), lambda i, r: (0, 0)),
            pl.BlockSpec((b * (ch // 3), b), lambda i, r: (0, 0)),
        ],
        out_specs=[
            pl.BlockSpec((b, 9, rps * ch, w),
                         lambda i, r: (0, 0, jnp.where(i == _N_ITERS, r, 0), 0)),
            pl.BlockSpec((n_sp, b * c), lambda i, r: (0, 0)),
        ],
        out_shape=[
            jax.ShapeDtypeStruct((b, 9, h, w), jnp.float32),
            jax.ShapeDtypeStruct((n_sp, b * c), jnp.float32),
        ],
        scratch_shapes=[
            pltpu.VMEM((n_sp, b * c), jnp.float32),
            pltpu.VMEM((n_sp, b * c), jnp.float32),
            pltpu.VMEM((n_sp, b), jnp.float32),
        ],
        compiler_params=pltpu.CompilerParams(
            dimension_semantics=("arbitrary", "arbitrary")),
    )(x, E, G, cbias, R, R2L, Gn8, Gd8)
    spf_out = spf_p.T.reshape(b, c, n_sp)
    return (q, x, spf_out, x)


# speedup vs baseline: 1.1956x; 1.0099x over previous
import functools
import math

import jax
import jax.numpy as jnp
import numpy as np
from jax.experimental import pallas as pl
from jax.experimental.pallas import tpu as pltpu

_N_SPIXELS = 256
_N_ITERS = 5
_NEG = -1e16
_L2E = 1.4426950408889634


def _cells_layout(h, w, n_spixels):
    nw = int(math.sqrt(n_spixels * w / h) + 0.5)
    nh = int(math.sqrt(n_spixels * h / w) + 0.5)
    cw = int(math.ceil(w / nw))
    ch = int(math.ceil(h / nh))
    return nh, nw, ch, cw


def _consts(h, w, nh, nw, ch, cw, b, c):
    j_of_l = np.minimum(np.arange(w) // cw, nw - 1)
    E = np.zeros((3, w, nw), np.float32)
    cbias = np.zeros((3, 1, w), np.float32)
    for t, dx in enumerate((-1, 0, 1)):
        jj = j_of_l + dx
        ok = (jj >= 0) & (jj < nw)
        jc = np.clip(jj, 0, nw - 1)
        E[t, np.arange(w), jc] = 1.0
        cbias[t, 0] = np.where(ok, 0.0, _NEG).astype(np.float32)
    G = np.ascontiguousarray(np.transpose(E, (0, 2, 1)))
    R = np.zeros((b, b * c), np.float32)
    for bi in range(b):
        R[bi, bi * c:(bi + 1) * c] = 1.0
    R2L = np.zeros((b * c, b), np.float32)
    for g in range(b * c):
        R2L[g, g // c] = _L2E
    Gn8 = np.zeros((b * c * (ch // 3), b * c), np.float32)
    for g in range(b * c * (ch // 3)):
        Gn8[g, g // (ch // 3)] = 1.0
    Gd8 = np.zeros((b * (ch // 3), b), np.float32)
    for g in range(b * (ch // 3)):
        Gd8[g, g // (ch // 3)] = 1.0
    return tuple(jnp.asarray(a) for a in (E, G, cbias, R, R2L, Gn8, Gd8))


def _scat(e_k, col):
    return jax.lax.dot_general(e_k, col, (((0,), (1,)), ((), ())),
                               preferred_element_type=jnp.float32)


def _mm(a, bmat):
    return jax.lax.dot_general(a, bmat, (((1,), (0,)), ((), ())),
                               preferred_element_type=jnp.float32)


def _collapse3(arr, g, ch, w):
    return arr.reshape(g, 3, ch // 3, w).sum(axis=1).reshape(g * (ch // 3), w)


def _ssn_body(x_ref, e_ref, g_ref, cbias_ref, rrep_ref, r2l_ref, gn8_ref,
              gd8_ref, q_ref, spfp_ref, spf_s, num_s, den_s,
              *, nh, nw, ch, b, c, n_iters, rows_per_step):
    i = pl.program_id(0)
    rb = pl.program_id(1)
    w = x_ref.shape[-1]
    bc = b * c
    XB = x_ref[...]

    @pl.when(jnp.logical_and(i == 0, rb == 0))
    def _zero():
        num_s[...] = jnp.zeros_like(num_s)
        den_s[...] = jnp.zeros_like(den_s)

    @pl.when(i == 0)
    def _init():
        for sub in range(rows_per_step):
            r = rb * rows_per_step + sub
            X20 = XB[:, :, sub * ch:(sub + 1) * ch, :].reshape(bc, ch, w)
            col8 = _collapse3(X20, bc, ch, w)
            cn = _mm(_scat(e_ref[1], col8), gn8_ref[...])
            num_s[pl.ds(r * nw, nw), :] += cn
            cnt = jnp.full((b, w), float(ch), jnp.float32)
            den_s[pl.ds(r * nw, nw), :] += _scat(e_ref[1], cnt)

    @pl.when(i > 0)
    def _iterate():
        rps = rows_per_step
        r0 = rb * rps
        maps = {}
        for off in range(rps + 2):
            rn = r0 + off - 1
            rbias = jnp.where(jnp.logical_and(rn >= 0, rn < nh),
                              jnp.float32(0.0), jnp.float32(_NEG))
            rp = jnp.clip(rn, 0, nh - 1)
            S = spf_s[pl.ds(rp * nw, nw), :]
            Saug = jnp.concatenate(
                [(2.0 * _L2E) * S, _mm(S * S, r2l_ref[...])], axis=1)
            for t_dx in range(3):
                MapA = jax.lax.dot_general(
                    Saug, g_ref[t_dx], (((0,), (0,)), ((), ())),
                    preferred_element_type=jnp.float32)
                M2 = MapA[:bc].reshape(b, c, 1, w)
                baserow = (cbias_ref[t_dx] + rbias) - MapA[bc:bc + b]
                maps[(off, t_dx)] = (M2, baserow)

        win_n = {}
        win_d = {}
        for sub in range(rps):
            X = XB[:, :, sub * ch:(sub + 1) * ch, :]
            xsqL = _L2E * jnp.sum(X * X, axis=1)
            nd = []
            for t_dy in range(3):
                for t_dx in range(3):
                    M2, baserow = maps[(sub + t_dy, t_dx)]
                    cr = jnp.sum(X * M2, axis=1)
                    nd.append((baserow[:, None, :] - xsqL) + cr)
            m = nd[0]
            for t in range(1, 9):
                m = jnp.maximum(m, nd[t])
            ex = [jnp.exp2(nd[t] - m) for t in range(9)]
            s = ex[0]
            for t in range(1, 9):
                s = s + ex[t]
            rs = 1.0 / s
            Xn = X * rs[:, None]

            @pl.when(i == n_iters)
            def _emit_q(sub=sub, ex=ex, rs=rs):
                for t in range(9):
                    q_ref[:, t, sub * ch:(sub + 1) * ch, :] = ex[t] * rs

            for t_dy in range(3):
                an = jnp.zeros((nw, bc * (ch // 3)), jnp.float32)
                ad = jnp.zeros((nw, b * (ch // 3)), jnp.float32)
                for t_dx in range(3):
                    k = t_dy * 3 + t_dx
                    P8 = _collapse3(ex[k][:, None] * Xn, bc, ch, w)
                    q8 = _collapse3(ex[k] * rs, b, ch, w)
                    an = an + _scat(e_ref[t_dx], P8)
                    ad = ad + _scat(e_ref[t_dx], q8)
                woff = sub + t_dy
                cn = _mm(an, gn8_ref[...])
                cd = _mm(ad, gd8_ref[...])
                win_n[woff] = cn if woff not in win_n else win_n[woff] + cn
                win_d[woff] = cd if woff not in win_d else win_d[woff] + cd

        interior_n = jnp.concatenate([win_n[o] for o in range(1, rps + 1)],
                                     axis=0)
        interior_d = jnp.concatenate([win_d[o] for o in range(1, rps + 1)],
                                     axis=0)
        num_s[pl.ds(r0 * nw, rps * nw), :] += interior_n
        den_s[pl.ds(r0 * nw, rps * nw), :] += interior_d
        rtop = jnp.clip(r0 - 1, 0, nh - 1)
        num_s[pl.ds(rtop * nw, nw), :] += win_n[0]
        den_s[pl.ds(rtop * nw, nw), :] += win_d[0]
        rbot = jnp.clip(r0 + rps, 0, nh - 1)
        num_s[pl.ds(rbot * nw, nw), :] += win_n[rps + 1]
        den_s[pl.ds(rbot * nw, nw), :] += win_d[rps + 1]

    @pl.when(rb == nh // rows_per_step - 1)
    def _finalize():
        den = den_s[...]
        den_bc = jax.lax.dot_general(
            den, rrep_ref[...], (((1,), (0,)), ((), ())),
            preferred_element_type=jnp.float32)
        denom = jnp.where(i == 0, jnp.maximum(den_bc, 1.0), den_bc + 1e-16)
        spf = num_s[...] / denom
        spf_s[...] = spf
        num_s[...] = jnp.zeros_like(num_s)
        den_s[...] = jnp.zeros_like(den_s)

        @pl.when(i == n_iters)
        def _emit_spf():
            spfp_ref[...] = spf


@jax.jit
def kernel(x):
    b, c, h, w = x.shape
    nh, nw, ch, cw = _cells_layout(h, w, _N_SPIXELS)
    assert nh * ch == h and nw * cw == w, "kernel assumes even cell tiling"
    assert ch % 3 == 0
    n_sp = nh * nw
    E, G, cbias, R, R2L, Gn8, Gd8 = _consts(h, w, nh, nw, ch, cw, b, c)
    rps = 8
    assert nh % rps == 0
    grid = (_N_ITERS + 1, nh // rps)
    body = functools.partial(_ssn_body, nh=nh, nw=nw, ch=ch, b=b, c=c,
                             n_iters=_N_ITERS, rows_per_step=rps)
    q, spf_p = pl.pallas_call(
        body,
        grid=grid,
        in_specs=[
            pl.BlockSpec((b, c, rps * ch, w), lambda i, r: (0, 0, r, 0)),
            pl.BlockSpec((3, w, nw), lambda i, r: (0, 0, 0)),
            pl.BlockSpec((3, nw, w), lambda i, r: (0, 0, 0)),
            pl.BlockSpec((3, 1, w), lambda i, r: (0, 0, 0)),
            pl.BlockSpec((b, b * c), lambda i, r: (0, 0)),
            pl.BlockSpec((b * c, b), lambda i, r: (0, 0)),
            pl.BlockSpec((b * c * (ch // 3), b * c), lambda i, r: (0, 0)),
            pl.BlockSpec((b * (ch // 3), b), lambda i, r: (0, 0)),
        ],
        out_specs=[
            pl.BlockSpec((b, 9, rps * ch, w),
                         lambda i, r: (0, 0, jnp.where(i == _N_ITERS, r, 0), 0)),
            pl.BlockSpec((n_sp, b * c), lambda i, r: (0, 0)),
        ],
        out_shape=[
            jax.ShapeDtypeStruct((b, 9, h, w), jnp.float32),
            jax.ShapeDtypeStruct((n_sp, b * c), jnp.float32),
        ],
        scratch_shapes=[
            pltpu.VMEM((n_sp, b * c), jnp.float32),
            pltpu.VMEM((n_sp, b * c), jnp.float32),
            pltpu.VMEM((n_sp, b), jnp.float32),
        ],
        compiler_params=pltpu.CompilerParams(
            dimension_semantics=("arbitrary", "arbitrary")),
    )(x, E, G, cbias, R, R2L, Gn8, Gd8)
    spf_out = spf_p.T.reshape(b, c, n_sp)
    return (q, x, spf_out, x)
